# Initial kernel scaffold; baseline (speedup 1.0000x reference)
import jax
import jax.numpy as jnp
from jax.experimental import pallas as pl


def kernel(node_embedding, nbr_idx):
    raise NotImplementedError("write your pallas kernel here")


# speedup vs baseline: 847.8065x; 847.8065x over previous
import jax
import jax.numpy as jnp
import numpy as np
from jax import lax
from jax.experimental import pallas as pl
from jax.experimental.pallas import tpu as pltpu
from jax.experimental.pallas import tpu_sc as plsc

B, AT, NBR, NFEAT = 2, 512, 16, 128
NM = NBR - 1
RPA = NBR * NM
NCHUNK = RPA // 16
NC, NS = 2, 16
NW = NC * NS
NATOMS = B * AT
APW = NATOMS // NW

_kidx = np.stack([np.delete(np.arange(NBR), i) for i in range(NBR)], axis=0)
_PAT = np.ascontiguousarray(_kidx.reshape(NCHUNK, 16).astype(np.int32))


def _sc_body(emb_hbm, nbr_hbm, pat_hbm, out_hbm, nbr_v, pat_v, idx_v, rows_v, sem):
    wid = lax.axis_index("s") * NC + lax.axis_index("c")
    base = wid * APW
    pltpu.sync_copy(pat_hbm, pat_v)
    pltpu.sync_copy(nbr_hbm.at[pl.ds(base * NBR, APW * NBR)], nbr_v)

    def atom_body(a, carry):
        off = a * NBR
        for t in range(NCHUNK):
            idx_v[pl.ds(t * 16, 16)] = plsc.load_gather(nbr_v, [pat_v[t, :] + off])
        pltpu.async_copy(emb_hbm.at[idx_v], rows_v, sem).wait()
        pltpu.sync_copy(rows_v, out_hbm.at[pl.ds((base + a) * RPA, RPA)])
        return carry

    lax.fori_loop(0, APW, atom_body, 0)


def kernel(node_embedding, nbr_idx):
    emb_flat = node_embedding.reshape(NATOMS, NFEAT)
    batch_off = (jnp.arange(B, dtype=jnp.int32) * AT)[:, None, None]
    nbr_glob = (nbr_idx.astype(jnp.int32) + batch_off).reshape(NATOMS * NBR)

    run = pl.kernel(
        _sc_body,
        out_type=jax.ShapeDtypeStruct((NATOMS * RPA, NFEAT), jnp.float32),
        mesh=plsc.VectorSubcoreMesh(core_axis_name="c", subcore_axis_name="s"),
        scratch_types=[
            pltpu.VMEM((APW * NBR,), jnp.int32),
            pltpu.VMEM((NCHUNK, 16), jnp.int32),
            pltpu.VMEM((RPA,), jnp.int32),
            pltpu.VMEM((RPA, NFEAT), jnp.float32),
            pltpu.SemaphoreType.DMA,
        ],
        compiler_params=pltpu.CompilerParams(needs_layout_passes=False),
    )
    out = run(emb_flat, nbr_glob, jnp.asarray(_PAT))
    return out.reshape(B, AT, NBR, NM, NFEAT)


# speedup vs baseline: 1372.9073x; 1.6194x over previous
import jax
import jax.numpy as jnp
from jax import lax
from jax.experimental import pallas as pl
from jax.experimental.pallas import tpu as pltpu
from jax.experimental.pallas import tpu_sc as plsc

B, AT, NBR, NFEAT = 2, 512, 16, 128
NM = NBR - 1
RPA = NBR * NM
NC, NS = 2, 16
NW = NC * NS
NATOMS = B * AT
APW = NATOMS // NW

_RUNS = []
for _i in range(NBR):
    if _i > 0:
        _RUNS.append((0, _i * NM, _i))
    if _i < NBR - 1:
        _RUNS.append((_i + 1, _i * NM + _i, NM - _i))


def _wait_runs(out_hbm, rows_v, wsem):
    for _, _, n in _RUNS:
        pltpu.make_async_copy(
            rows_v.at[pl.ds(0, n)], out_hbm.at[pl.ds(0, n)], wsem
        ).wait()


def _sc_body(emb_hbm, nbr_hbm, out_hbm, nbr_v, rows_v, gsem, wsem):
    wid = lax.axis_index("s") * NC + lax.axis_index("c")
    base = wid * APW
    pltpu.sync_copy(nbr_hbm.at[pl.ds(base * NBR, APW * NBR)], nbr_v)
    pltpu.async_copy(emb_hbm.at[nbr_v], rows_v, gsem).wait()

    def atom_body(a, carry):
        arow = a * NBR
        obase = (base + a) * RPA
        for src, dst, n in _RUNS:
            pltpu.async_copy(
                rows_v.at[pl.ds(arow + src, n)],
                out_hbm.at[pl.ds(obase + dst, n)],
                wsem,
            )

        @pl.when(a >= 1)
        def _():
            _wait_runs(out_hbm, rows_v, wsem)

        return carry

    lax.fori_loop(0, APW, atom_body, 0)
    _wait_runs(out_hbm, rows_v, wsem)


def kernel(node_embedding, nbr_idx):
    emb_flat = node_embedding.reshape(NATOMS, NFEAT)
    batch_off = (jnp.arange(B, dtype=jnp.int32) * AT)[:, None, None]
    nbr_glob = (nbr_idx.astype(jnp.int32) + batch_off).reshape(NATOMS * NBR)

    run = pl.kernel(
        _sc_body,
        out_type=jax.ShapeDtypeStruct((NATOMS * RPA, NFEAT), jnp.float32),
        mesh=plsc.VectorSubcoreMesh(core_axis_name="c", subcore_axis_name="s"),
        scratch_types=[
            pltpu.VMEM((APW * NBR,), jnp.int32),
            pltpu.VMEM((APW * NBR, NFEAT), jnp.float32),
            pltpu.SemaphoreType.DMA,
            pltpu.SemaphoreType.DMA,
        ],
        compiler_params=pltpu.CompilerParams(
            needs_layout_passes=False, use_tc_tiling_on_sc=False
        ),
    )
    out = run(emb_flat, nbr_glob)
    return out.reshape(B, AT, NBR, NM, NFEAT)


# speedup vs baseline: 1550.3427x; 1.1292x over previous
import jax
import jax.numpy as jnp
from jax import lax
from jax.experimental import pallas as pl
from jax.experimental.pallas import tpu as pltpu
from jax.experimental.pallas import tpu_sc as plsc

B, AT, NBR, NFEAT = 2, 512, 16, 128
NM = NBR - 1
RPA = NBR * NM
NC, NS = 2, 16
NW = NC * NS
NATOMS = B * AT
APW = NATOMS // NW

_RUNS = []
for _i in range(NBR):
    if _i > 0:
        _RUNS.append((0, _i * NM, _i))
    if _i < NBR - 1:
        _RUNS.append((_i + 1, _i * NM + _i, NM - _i))


def _sc_body(emb_hbm, nbr_hbm, out_hbm, nbr_v, rows_v, gsem, wsem):
    wid = lax.axis_index("s") * NC + lax.axis_index("c")
    base = wid * APW
    pltpu.sync_copy(nbr_hbm.at[pl.ds(base, APW)], nbr_v)
    for a in range(APW):
        pltpu.async_copy(emb_hbm.at[nbr_v.at[a]], rows_v.at[a], gsem)
    for a in range(APW):
        pltpu.make_async_copy(emb_hbm.at[nbr_v.at[a]], rows_v.at[a], gsem).wait()
    for src, dst, n in _RUNS:
        pltpu.async_copy(
            rows_v.at[:, pl.ds(src, n)],
            out_hbm.at[pl.ds(base, APW), pl.ds(dst, n)],
            wsem,
        )
    for src, dst, n in _RUNS:
        pltpu.make_async_copy(
            rows_v.at[:, pl.ds(src, n)],
            out_hbm.at[pl.ds(base, APW), pl.ds(dst, n)],
            wsem,
        ).wait()


def kernel(node_embedding, nbr_idx):
    emb_flat = node_embedding.reshape(NATOMS, NFEAT)
    batch_off = (jnp.arange(B, dtype=jnp.int32) * AT)[:, None, None]
    nbr_glob = (nbr_idx.astype(jnp.int32) + batch_off).reshape(NATOMS, NBR)

    run = pl.kernel(
        _sc_body,
        out_type=jax.ShapeDtypeStruct((NATOMS, RPA, NFEAT), jnp.float32),
        mesh=plsc.VectorSubcoreMesh(core_axis_name="c", subcore_axis_name="s"),
        scratch_types=[
            pltpu.VMEM((APW, NBR), jnp.int32),
            pltpu.VMEM((APW, NBR, NFEAT), jnp.float32),
            pltpu.SemaphoreType.DMA,
            pltpu.SemaphoreType.DMA,
        ],
        compiler_params=pltpu.CompilerParams(
            needs_layout_passes=False, use_tc_tiling_on_sc=False
        ),
    )
    out = run(emb_flat, nbr_glob)
    return out.reshape(B, AT, NBR, NM, NFEAT)
